# baseline (speedup 1.0000x reference)
import functools

import jax
import jax.numpy as jnp
from jax import lax
from jax.experimental import pallas as pl
from jax.experimental.pallas import tpu as pltpu
from jax.experimental.pallas import tpu_sc as plsc

DIM = 32
BATCH = 16384
NUM_TABLES = 4
ICHUNK = 128
LANES = 16
BLK = 128


@functools.cache
def _build_kernel():
    info = plsc.get_sparse_core_info()
    nc, ns = info.num_cores, info.num_subcores
    nw = nc * ns
    b_per_w = BATCH // nw
    n_chunks = b_per_w // ICHUNK
    n_rows = NUM_TABLES * n_chunks
    mesh = plsc.VectorSubcoreMesh(core_axis_name="c", subcore_axis_name="s")

    @functools.partial(
        pl.kernel,
        mesh=mesh,
        out_type=jax.ShapeDtypeStruct((BATCH, NUM_TABLES * DIM), jnp.float32),
        compiler_params=pltpu.CompilerParams(needs_layout_passes=False),
        scratch_types=[
            pltpu.VMEM((NUM_TABLES * (BATCH // nw) // ICHUNK, ICHUNK), jnp.int32),
            pltpu.VMEM((NUM_TABLES * (BATCH // nw) // ICHUNK, ICHUNK), jnp.int32),
            pltpu.VMEM((NUM_TABLES * (BATCH // nw) // ICHUNK, ICHUNK), jnp.int32),
            pltpu.VMEM((NUM_TABLES * ICHUNK, BLK), jnp.float32),
            pltpu.VMEM((ICHUNK, NUM_TABLES * DIM), jnp.float32),
            pltpu.SemaphoreType.DMA,
            pltpu.SemaphoreType.DMA,
        ],
    )
    def merged_embed(
        x0, x1, x2, x3, w0, w1, w2, w3, out,
        idx_v, top_v, sub_v, tiles_v, comb_v, isem, gsem,
    ):
        wid = lax.axis_index("s") * nc + lax.axis_index("c")
        base = wid * b_per_w
        xs = (x0, x1, x2, x3)
        ws = (w0, w1, w2, w3)

        icopies = []
        for i in range(NUM_TABLES):
            for c in range(n_chunks):
                icopies.append(
                    pltpu.async_copy(
                        xs[i].at[pl.ds(base + c * ICHUNK, ICHUNK)],
                        idx_v.at[i * n_chunks + c],
                        isem,
                    )
                )
        for cp in icopies:
            cp.wait()

        iota = lax.iota(jnp.int32, LANES)

        @pl.loop(0, n_chunks)
        def chunk_body(c):
            for i in range(NUM_TABLES):

                @plsc.parallel_loop(0, ICHUNK // LANES)
                def split(g, i=i):
                    r = i * n_chunks + c
                    v = idx_v[r, pl.ds(g * LANES, LANES)]
                    top_v[r, pl.ds(g * LANES, LANES)] = v >> 2
                    sub_v[r, pl.ds(g * LANES, LANES)] = (v & 3) << 5

            gathers = []
            for i in range(NUM_TABLES):
                gathers.append(
                    pltpu.async_copy(
                        ws[i].at[top_v.at[i * n_chunks + c]],
                        tiles_v.at[pl.ds(i * ICHUNK, ICHUNK)],
                        gsem,
                    )
                )
            for cp in gathers:
                cp.wait()

            for i in range(NUM_TABLES):

                @plsc.parallel_loop(0, ICHUNK // LANES)
                def extract(g, i=i):
                    sub16 = sub_v[i * n_chunks + c, pl.ds(g * LANES, LANES)]
                    rows16 = g * LANES + iota
                    trow16 = i * ICHUNK + rows16
                    for col in range(DIM):
                        vals = plsc.load_gather(tiles_v, [trow16, sub16 + col])
                        plsc.store_scatter(
                            comb_v,
                            [rows16, jnp.full((LANES,), i * DIM + col, jnp.int32)],
                            vals,
                        )

            pltpu.sync_copy(
                comb_v,
                out.at[pl.ds(base + c * ICHUNK, ICHUNK), :],
            )

    return merged_embed


def kernel(x0, x1, x2, x3, W0, W1, W2, W3):
    k = _build_kernel()
    out = k(
        x0.astype(jnp.int32),
        x1.astype(jnp.int32),
        x2.astype(jnp.int32),
        x3.astype(jnp.int32),
        W0.reshape(-1, NUM_TABLES * DIM),
        W1.reshape(-1, NUM_TABLES * DIM),
        W2.reshape(-1, NUM_TABLES * DIM),
        W3.reshape(-1, NUM_TABLES * DIM),
    )
    return out.reshape(BATCH, NUM_TABLES * DIM)
